# baseline (speedup 1.0000x reference)
import functools

import jax
import jax.numpy as jnp
from jax import lax
from jax.experimental import pallas as pl
from jax.experimental.pallas import tpu as pltpu
from jax.experimental.pallas import tpu_sc as plsc

_BATCH = 16384
_FIELDS = 26
_V = 1000000
_NC = 2
_NS = 16
_NW = _NC * _NS
_ROWS = _BATCH // _NW
_CROWS = 64
_NCH = _ROWS // _CROWS
_CE = _CROWS * _FIELDS

_CHUNK = 62504
_TAIL = _V - 15 * _CHUNK
_SCHUNK = 13312
_NFULL = _CHUNK // _SCHUNK
_BTAIL = _CHUNK - _NFULL * _SCHUNK
_TTAIL = _TAIL - _NFULL * _SCHUNK

_mesh = plsc.VectorSubcoreMesh(core_axis_name="c", subcore_axis_name="s")


@functools.partial(
    pl.kernel,
    mesh=_mesh,
    out_type=jax.ShapeDtypeStruct((_BATCH, _FIELDS), jnp.float32),
    scratch_types=[
        pltpu.VMEM((_CROWS, _FIELDS), jnp.int32),
        pltpu.VMEM((_ROWS * _FIELDS,), jnp.int32),
        pltpu.VMEM((_CE,), jnp.float32),
        pltpu.VMEM((_CE,), jnp.float32),
        pltpu.VMEM((_CROWS, _FIELDS), jnp.float32),
        pltpu.VMEM((_SCHUNK,), jnp.float32),
        pltpu.VMEM((_SCHUNK,), jnp.float32),
        pltpu.VMEM_SHARED((_V,), jnp.float32),
        pltpu.SemaphoreType.DMA,
        pltpu.SemaphoreType.DMA,
        pltpu.SemaphoreType.DMA,
        pltpu.SemaphoreType.DMA,
    ],
)
def _gather_sc(idx_hbm, table_hbm, out_hbm, idx2_v, idx_v, vals_a, vals_b,
               vals2_v, stage_a, stage_b, table_sp, sem1, sem2, gsem_a,
               gsem_b):
    s = lax.axis_index("s")
    wid = s * _NC + lax.axis_index("c")
    row0 = wid * _ROWS
    stage = (stage_a, stage_b)

    def _flatten_chunk(ch):
        pltpu.sync_copy(idx_hbm.at[pl.ds(row0 + ch * _CROWS, _CROWS)], idx2_v)

        def _rows8(g, _):
            r = g * 8
            for k in range(8):
                fo = (ch * _CROWS + r + k) * _FIELDS
                a = idx2_v[r + k, pl.ds(0, 16)]
                b = idx2_v[r + k, pl.ds(10, 16)]
                idx_v[pl.ds(fo, 16)] = a
                idx_v[pl.ds(fo + 10, 16)] = b
            return _

        lax.fori_loop(0, _CROWS // 8, _rows8, None)

    def _stage_pipeline(pieces):
        np_ = len(pieces)
        off0, sz0 = pieces[0]
        h1 = pltpu.async_copy(table_hbm.at[pl.ds(off0, sz0)],
                              stage[0].at[pl.ds(0, sz0)], sem1)
        ch = 0
        for j, (off, sz) in enumerate(pieces):
            h1.wait()
            h2 = pltpu.async_copy(stage[j % 2].at[pl.ds(0, sz)],
                                  table_sp.at[pl.ds(off, sz)], sem2)
            if j + 1 < np_:
                off_n, sz_n = pieces[j + 1]
                h1 = pltpu.async_copy(table_hbm.at[pl.ds(off_n, sz_n)],
                                      stage[(j + 1) % 2].at[pl.ds(0, sz_n)],
                                      sem1)
            if ch < _NCH:
                _flatten_chunk(ch)
                ch += 1
            h2.wait()
        while ch < _NCH:
            _flatten_chunk(ch)
            ch += 1

    @pl.when(s < _NS - 1)
    def _stage_body():
        base = pl.multiple_of(s * _CHUNK, 8)
        pieces = [(pl.multiple_of(base + j * _SCHUNK, 8), _SCHUNK)
                  for j in range(_NFULL)]
        pieces.append((pl.multiple_of(base + _NFULL * _SCHUNK, 8), _BTAIL))
        _stage_pipeline(pieces)

    @pl.when(s == _NS - 1)
    def _stage_tail():
        pieces = [(15 * _CHUNK + j * _SCHUNK, _SCHUNK)
                  for j in range(_NFULL)]
        pieces.append((15 * _CHUNK + _NFULL * _SCHUNK, _TTAIL))
        _stage_pipeline(pieces)

    plsc.subcore_barrier()

    vals = (vals_a, vals_b)
    gsem = (gsem_a, gsem_b)

    def _start_gather(ch):
        return pltpu.async_copy(
            table_sp.at[idx_v.at[pl.ds(ch * _CE, _CE)]], vals[ch % 2],
            gsem[ch % 2])

    def _unflat_store(ch):
        vb = vals[ch % 2]

        def _rows8(g, _):
            r = g * 8
            for k in range(8):
                fo = (r + k) * _FIELDS
                a = vb[pl.ds(fo, 16)]
                b = vb[pl.ds(fo + 10, 16)]
                vals2_v[r + k, pl.ds(0, 16)] = a
                vals2_v[r + k, pl.ds(10, 16)] = b
            return _

        lax.fori_loop(0, _CROWS // 8, _rows8, None)
        pltpu.sync_copy(vals2_v,
                        out_hbm.at[pl.ds(row0 + ch * _CROWS, _CROWS)])

    g = _start_gather(0)
    for ch in range(_NCH):
        g_next = _start_gather(ch + 1) if ch + 1 < _NCH else None
        g.wait()
        _unflat_store(ch)
        g = g_next


def kernel(ind, scales):
    if ind.dtype != jnp.int32:
        ind = ind.astype(jnp.int32)
    return _gather_sc(ind, scales)


# speedup vs baseline: 1.2644x; 1.2644x over previous
import functools

import jax
import jax.numpy as jnp
from jax import lax
from jax.experimental import pallas as pl
from jax.experimental.pallas import tpu as pltpu
from jax.experimental.pallas import tpu_sc as plsc

_BATCH = 16384
_FIELDS = 26
_V = 1000000
_NC = 2
_NS = 16
_NW = _NC * _NS
_COLS = _BATCH // _NW
_HCOLS = _COLS // 2
_HE = _FIELDS * _HCOLS
_VPR = _HCOLS // 16

_CHUNK = 62504
_TAIL = _V - 15 * _CHUNK
_SCHUNK = 8192
_NFULL = _CHUNK // _SCHUNK
_BTAIL = _CHUNK - _NFULL * _SCHUNK
_TTAIL = _TAIL - _NFULL * _SCHUNK

_mesh = plsc.VectorSubcoreMesh(core_axis_name="c", subcore_axis_name="s")


@functools.partial(
    pl.kernel,
    mesh=_mesh,
    out_type=jax.ShapeDtypeStruct((_FIELDS, _BATCH), jnp.float32),
    scratch_types=[
        pltpu.VMEM((_FIELDS, _HCOLS), jnp.int32),
        pltpu.VMEM((_HE,), jnp.int32),
        pltpu.VMEM((_HE,), jnp.int32),
        pltpu.VMEM((_HE,), jnp.float32),
        pltpu.VMEM((_HE,), jnp.float32),
        pltpu.VMEM((_FIELDS, _HCOLS), jnp.float32),
        pltpu.VMEM((_SCHUNK,), jnp.float32),
        pltpu.VMEM((_SCHUNK,), jnp.float32),
        pltpu.VMEM_SHARED((_V,), jnp.float32),
        pltpu.SemaphoreType.DMA,
        pltpu.SemaphoreType.DMA,
        pltpu.SemaphoreType.DMA,
        pltpu.SemaphoreType.DMA,
    ],
)
def _gather_sc(idx_hbm, table_hbm, out_hbm, idx2_v, idx_a, idx_b, vals_a,
               vals_b, vals2_v, stage_a, stage_b, table_sp, sem1, sem2,
               gsem_a, gsem_b):
    s = lax.axis_index("s")
    wid = s * _NC + lax.axis_index("c")
    col0 = wid * _COLS
    stage = (stage_a, stage_b)
    idx_flat = (idx_a, idx_b)
    vals = (vals_a, vals_b)
    gsem = (gsem_a, gsem_b)

    def _flatten_half(hb):
        pltpu.sync_copy(idx_hbm.at[:, pl.ds(col0 + hb * _HCOLS, _HCOLS)],
                        idx2_v)
        dst = idx_flat[hb]

        def _row(f, _):
            for j in range(_VPR):
                dst[pl.ds(f * _HCOLS + j * 16, 16)] = \
                    idx2_v[f, pl.ds(j * 16, 16)]
            return _

        lax.fori_loop(0, _FIELDS, _row, None)

    def _stage_pipeline(pieces):
        np_ = len(pieces)
        off0, sz0 = pieces[0]
        h1 = pltpu.async_copy(table_hbm.at[pl.ds(off0, sz0)],
                              stage[0].at[pl.ds(0, sz0)], sem1)
        done = 0
        for j, (off, sz) in enumerate(pieces):
            h1.wait()
            h2 = pltpu.async_copy(stage[j % 2].at[pl.ds(0, sz)],
                                  table_sp.at[pl.ds(off, sz)], sem2)
            if j + 1 < np_:
                off_n, sz_n = pieces[j + 1]
                h1 = pltpu.async_copy(table_hbm.at[pl.ds(off_n, sz_n)],
                                      stage[(j + 1) % 2].at[pl.ds(0, sz_n)],
                                      sem1)
            if done < 2 and j % 4 == 1:
                _flatten_half(done)
                done += 1
            h2.wait()
        while done < 2:
            _flatten_half(done)
            done += 1

    @pl.when(s < _NS - 1)
    def _stage_body():
        base = pl.multiple_of(s * _CHUNK, 8)
        pieces = [(pl.multiple_of(base + j * _SCHUNK, 8), _SCHUNK)
                  for j in range(_NFULL)]
        pieces.append((pl.multiple_of(base + _NFULL * _SCHUNK, 8), _BTAIL))
        _stage_pipeline(pieces)

    @pl.when(s == _NS - 1)
    def _stage_tail():
        pieces = [(15 * _CHUNK + j * _SCHUNK, _SCHUNK)
                  for j in range(_NFULL)]
        pieces.append((15 * _CHUNK + _NFULL * _SCHUNK, _TTAIL))
        _stage_pipeline(pieces)

    plsc.subcore_barrier()

    g0 = pltpu.async_copy(table_sp.at[idx_a], vals_a, gsem_a)
    g1 = pltpu.async_copy(table_sp.at[idx_b], vals_b, gsem_b)

    for hb, g in ((0, g0), (1, g1)):
        g.wait()
        vb = vals[hb]

        def _row(f, _):
            for j in range(_VPR):
                vals2_v[f, pl.ds(j * 16, 16)] = \
                    vb[pl.ds(f * _HCOLS + j * 16, 16)]
            return _

        lax.fori_loop(0, _FIELDS, _row, None)
        pltpu.sync_copy(vals2_v,
                        out_hbm.at[:, pl.ds(col0 + hb * _HCOLS, _HCOLS)])


def kernel(ind, scales):
    if ind.dtype != jnp.int32:
        ind = ind.astype(jnp.int32)
    return _gather_sc(ind.T, scales).T


# speedup vs baseline: 1.2662x; 1.0014x over previous
import functools

import jax
import jax.numpy as jnp
from jax import lax
from jax.experimental import pallas as pl
from jax.experimental.pallas import tpu as pltpu
from jax.experimental.pallas import tpu_sc as plsc

_BATCH = 16384
_FIELDS = 26
_V = 1000000
_NC = 2
_NS = 16
_NW = _NC * _NS
_COLS = _BATCH // _NW
_HCOLS = _COLS // 2
_HE = _FIELDS * _HCOLS
_VPR = _HCOLS // 16

_CHUNK = 62504
_TAIL = _V - 15 * _CHUNK
_SCHUNK = 8192
_NFULL = _CHUNK // _SCHUNK
_BTAIL = _CHUNK - _NFULL * _SCHUNK
_TTAIL = _TAIL - _NFULL * _SCHUNK

_mesh = plsc.VectorSubcoreMesh(core_axis_name="c", subcore_axis_name="s")


@functools.partial(
    pl.kernel,
    mesh=_mesh,
    out_type=jax.ShapeDtypeStruct((_FIELDS, _BATCH), jnp.float32),
    scratch_types=[
        pltpu.VMEM((_FIELDS, _HCOLS), jnp.int32),
        pltpu.VMEM((_HE,), jnp.int32),
        pltpu.VMEM((_HE,), jnp.int32),
        pltpu.VMEM((_HE,), jnp.float32),
        pltpu.VMEM((_HE,), jnp.float32),
        pltpu.VMEM((_FIELDS, _HCOLS), jnp.float32),
        pltpu.VMEM((_SCHUNK,), jnp.float32),
        pltpu.VMEM((_SCHUNK,), jnp.float32),
        pltpu.VMEM_SHARED((_V,), jnp.float32),
        pltpu.SemaphoreType.DMA,
        pltpu.SemaphoreType.DMA,
        pltpu.SemaphoreType.DMA,
        pltpu.SemaphoreType.DMA,
    ],
)
def _gather_sc(idx_hbm, table_hbm, out_hbm, idx2_v, idx_a, idx_b, vals_a,
               vals_b, vals2_v, stage_a, stage_b, table_sp, sem1, sem2,
               gsem_a, gsem_b):
    s = lax.axis_index("s")
    wid = s * _NC + lax.axis_index("c")
    col0 = wid * _COLS
    stage = (stage_a, stage_b)
    idx_flat = (idx_a, idx_b)
    vals = (vals_a, vals_b)
    gsem = (gsem_a, gsem_b)

    def _flatten_half(hb):
        pltpu.sync_copy(idx_hbm.at[:, pl.ds(col0 + hb * _HCOLS, _HCOLS)],
                        idx2_v)
        dst = idx_flat[hb]

        def _row(f, _):
            for j in range(_VPR):
                dst[pl.ds(f * _HCOLS + j * 16, 16)] = \
                    idx2_v[f, pl.ds(j * 16, 16)]
            return _

        lax.fori_loop(0, _FIELDS, _row, None)

    def _stage_pipeline(pieces):
        np_ = len(pieces)
        off0, sz0 = pieces[0]
        h1 = pltpu.async_copy(table_hbm.at[pl.ds(off0, sz0)],
                              stage[0].at[pl.ds(0, sz0)], sem1)
        done = 0
        for j, (off, sz) in enumerate(pieces):
            h1.wait()
            h2 = pltpu.async_copy(stage[j % 2].at[pl.ds(0, sz)],
                                  table_sp.at[pl.ds(off, sz)], sem2)
            if j + 1 < np_:
                off_n, sz_n = pieces[j + 1]
                h1 = pltpu.async_copy(table_hbm.at[pl.ds(off_n, sz_n)],
                                      stage[(j + 1) % 2].at[pl.ds(0, sz_n)],
                                      sem1)
            if done < 2 and j % 4 == 1:
                _flatten_half(done)
                done += 1
            h2.wait()
        while done < 2:
            _flatten_half(done)
            done += 1

    @pl.when(s < _NS - 1)
    def _stage_body():
      with jax.named_scope("stage"):
        base = pl.multiple_of(s * _CHUNK, 8)
        pieces = [(pl.multiple_of(base + j * _SCHUNK, 8), _SCHUNK)
                  for j in range(_NFULL)]
        pieces.append((pl.multiple_of(base + _NFULL * _SCHUNK, 8), _BTAIL))
        _stage_pipeline(pieces)

    @pl.when(s == _NS - 1)
    def _stage_tail():
        pieces = [(15 * _CHUNK + j * _SCHUNK, _SCHUNK)
                  for j in range(_NFULL)]
        pieces.append((15 * _CHUNK + _NFULL * _SCHUNK, _TTAIL))
        _stage_pipeline(pieces)

    with jax.named_scope("barrier"):
        plsc.subcore_barrier()

    g0 = pltpu.async_copy(table_sp.at[idx_a], vals_a, gsem_a)
    g1 = pltpu.async_copy(table_sp.at[idx_b], vals_b, gsem_b)

    for hb, g in ((0, g0), (1, g1)):
        with jax.named_scope(f"gwait{hb}"):
            g.wait()
        vb = vals[hb]

        def _row(f, _):
            for j in range(_VPR):
                vals2_v[f, pl.ds(j * 16, 16)] = \
                    vb[pl.ds(f * _HCOLS + j * 16, 16)]
            return _

        lax.fori_loop(0, _FIELDS, _row, None)
        pltpu.sync_copy(vals2_v,
                        out_hbm.at[:, pl.ds(col0 + hb * _HCOLS, _HCOLS)])


def kernel(ind, scales):
    if ind.dtype != jnp.int32:
        ind = ind.astype(jnp.int32)
    return _gather_sc(ind.T, scales).T


# speedup vs baseline: 1.3194x; 1.0421x over previous
import functools

import jax
import jax.numpy as jnp
from jax import lax
from jax.experimental import pallas as pl
from jax.experimental.pallas import tpu as pltpu
from jax.experimental.pallas import tpu_sc as plsc

_BATCH = 16384
_FIELDS = 26
_V = 1000000
_NC = 2
_NS = 16
_NW = _NC * _NS
_COLS = _BATCH // _NW
_HCOLS = _COLS // 2
_HE = _FIELDS * _HCOLS
_VPR = _HCOLS // 16

_CHUNK = 62504
_TAIL = _V - 15 * _CHUNK
_SCHUNK = 4096
_NFULL = _CHUNK // _SCHUNK
_BTAIL = _CHUNK - _NFULL * _SCHUNK
_TTAIL = _TAIL - _NFULL * _SCHUNK
_RING = 4

_mesh = plsc.VectorSubcoreMesh(core_axis_name="c", subcore_axis_name="s")


@functools.partial(
    pl.kernel,
    mesh=_mesh,
    out_type=jax.ShapeDtypeStruct((_FIELDS, _BATCH), jnp.float32),
    scratch_types=[
        pltpu.VMEM((_FIELDS, _HCOLS), jnp.int32),
        pltpu.VMEM((_HE,), jnp.int32),
        pltpu.VMEM((_HE,), jnp.int32),
        pltpu.VMEM((_HE,), jnp.float32),
        pltpu.VMEM((_HE,), jnp.float32),
        pltpu.VMEM((_FIELDS, _HCOLS), jnp.float32),
        pltpu.VMEM((_SCHUNK,), jnp.float32),
        pltpu.VMEM((_SCHUNK,), jnp.float32),
        pltpu.VMEM((_SCHUNK,), jnp.float32),
        pltpu.VMEM((_SCHUNK,), jnp.float32),
        pltpu.VMEM_SHARED((_V,), jnp.float32),
        pltpu.SemaphoreType.DMA,
        pltpu.SemaphoreType.DMA,
        pltpu.SemaphoreType.DMA,
        pltpu.SemaphoreType.DMA,
    ],
)
def _gather_sc(idx_hbm, table_hbm, out_hbm, idx2_v, idx_a, idx_b, vals_a,
               vals_b, vals2_v, stage_0, stage_1, stage_2, stage_3, table_sp,
               sem1, sem2, gsem_a, gsem_b):
    s = lax.axis_index("s")
    wid = s * _NC + lax.axis_index("c")
    col0 = wid * _COLS
    stage = (stage_0, stage_1, stage_2, stage_3)
    idx_flat = (idx_a, idx_b)
    vals = (vals_a, vals_b)
    gsem = (gsem_a, gsem_b)

    def _flatten_half(hb):
        pltpu.sync_copy(idx_hbm.at[:, pl.ds(col0 + hb * _HCOLS, _HCOLS)],
                        idx2_v)
        dst = idx_flat[hb]

        def _row(f, _):
            for j in range(_VPR):
                dst[pl.ds(f * _HCOLS + j * 16, 16)] = \
                    idx2_v[f, pl.ds(j * 16, 16)]
            return _

        lax.fori_loop(0, _FIELDS, _row, None)

    def _stage_pipeline(pieces):
        np_ = len(pieces)

        def _fire_h1(j):
            off, sz = pieces[j]
            return pltpu.async_copy(table_hbm.at[pl.ds(off, sz)],
                                    stage[j % _RING].at[pl.ds(0, sz)], sem1)

        h1 = [_fire_h1(j) for j in range(min(_RING, np_))]
        h2 = [None] * _RING
        done = 0
        for j, (off, sz) in enumerate(pieces):
            slot = j % _RING
            h1[slot].wait()
            h2[slot] = pltpu.async_copy(stage[slot].at[pl.ds(0, sz)],
                                        table_sp.at[pl.ds(off, sz)], sem2)
            if done < 2 and j in (1, 5):
                _flatten_half(done)
                done += 1
            if j + _RING < np_:
                h2[slot].wait()
                h1[slot] = _fire_h1(j + _RING)
                h2[slot] = None
        for slot in range(_RING):
            if h2[slot] is not None:
                h2[slot].wait()
        while done < 2:
            _flatten_half(done)
            done += 1

    @pl.when(s < _NS - 1)
    def _stage_body():
      with jax.named_scope("stage"):
        base = pl.multiple_of(s * _CHUNK, 8)
        pieces = [(pl.multiple_of(base + j * _SCHUNK, 8), _SCHUNK)
                  for j in range(_NFULL)]
        pieces.append((pl.multiple_of(base + _NFULL * _SCHUNK, 8), _BTAIL))
        _stage_pipeline(pieces)

    @pl.when(s == _NS - 1)
    def _stage_tail():
        pieces = [(15 * _CHUNK + j * _SCHUNK, _SCHUNK)
                  for j in range(_NFULL)]
        pieces.append((15 * _CHUNK + _NFULL * _SCHUNK, _TTAIL))
        _stage_pipeline(pieces)

    with jax.named_scope("barrier"):
        plsc.subcore_barrier()

    g0 = pltpu.async_copy(table_sp.at[idx_a], vals_a, gsem_a)
    g1 = pltpu.async_copy(table_sp.at[idx_b], vals_b, gsem_b)

    for hb, g in ((0, g0), (1, g1)):
        with jax.named_scope(f"gwait{hb}"):
            g.wait()
        vb = vals[hb]

        def _row(f, _):
            for j in range(_VPR):
                vals2_v[f, pl.ds(j * 16, 16)] = \
                    vb[pl.ds(f * _HCOLS + j * 16, 16)]
            return _

        lax.fori_loop(0, _FIELDS, _row, None)
        pltpu.sync_copy(vals2_v,
                        out_hbm.at[:, pl.ds(col0 + hb * _HCOLS, _HCOLS)])


def kernel(ind, scales):
    if ind.dtype != jnp.int32:
        ind = ind.astype(jnp.int32)
    return _gather_sc(ind.T, scales).T
